# baseline (speedup 1.0000x reference)
import functools

import jax
import jax.numpy as jnp
from jax import lax
from jax.experimental import pallas as pl
from jax.experimental.pallas import tpu as pltpu
from jax.experimental.pallas import tpu_sc as plsc

_EOS = 49407
_L = 16
_NW = 32


def _worker_id():
    return lax.axis_index("s") * 2 + lax.axis_index("c")


def _sc_body(ctx_hbm, ids_hbm, tok_hbm, pos_hbm, out_hbm,
             pos_t, buf, midb, idsrow, idxl, idxl2, sem, sem_out):
    seq, emb = 77, 768
    nchunk = emb // _L
    rows_per_w = ids_hbm.shape[0] // _NW

    wid = _worker_id()

    pltpu.sync_copy(pos_hbm, pos_t)

    def build_idx(cbp16):
        idxl[pl.ds(0, _L)] = idsrow[pl.ds(0, _L)]
        for off in (16, 32, 48):
            a = idsrow[pl.ds(off, _L)]
            sh = idsrow[pl.ds(off - 16, _L)]
            posv = lax.iota(jnp.int32, _L) + off
            idxl[pl.ds(off, _L)] = jnp.where(posv < cbp16, a, sh)
        a = idsrow[pl.ds(61, _L)]
        sh = idsrow[pl.ds(45, _L)]
        posv = lax.iota(jnp.int32, _L) + 61
        idxl2[...] = jnp.where(posv < cbp16, a, sh)

    def fire_gather():
        cp1 = pltpu.async_copy(tok_hbm.at[idxl], buf.at[pl.ds(0, 64)], sem)
        cp2 = pltpu.async_copy(tok_hbm.at[idxl2], buf.at[pl.ds(61, _L)], sem)
        return cp1, cp2

    def row_body(i, carry):
        b = wid * rows_per_w + i
        pltpu.sync_copy(ids_hbm.at[b], idsrow)

        m = jnp.full((_L,), 127.0, jnp.float32)
        for k in range(5):
            v = idsrow[pl.ds(_L * k, _L)]
            posv = (lax.iota(jnp.int32, _L) + _L * k).astype(jnp.float32)
            m = jnp.minimum(m, jnp.where(v == _EOS, posv, 127.0))
        m_s = jnp.min(m)
        eos = jnp.where(m_s >= 127.0, 0, m_s.astype(jnp.int32))

        cbp_raw = jnp.max(idsrow[pl.ds(96, _L)].astype(jnp.float32)).astype(jnp.int32)
        cbp_spec = jnp.minimum(jnp.where(cbp_raw == -1, eos, cbp_raw), seq - 16)
        eos_cbp = jnp.minimum(eos, seq - 16)

        @pl.when(i > 0)
        def _():
            pltpu.make_async_copy(buf, out_hbm.at[b], sem_out).wait()

        build_idx(cbp_spec + 16)
        cp1, cp2 = fire_gather()

        def wscan(acc):
            def wloop(j, a):
                r = a
                for c in range(nchunk):
                    v = midb[j, pl.ds(c * _L, _L)]
                    r = jnp.maximum(r, jnp.where(v != 0.0, 1.0, 0.0))
                return r
            return lax.fori_loop(0, 8, wloop, acc)
        pltpu.sync_copy(ctx_hbm.at[b, pl.ds(0, 8)], midb)
        accv = wscan(jnp.zeros((_L,), jnp.float32))
        pltpu.sync_copy(ctx_hbm.at[b, pl.ds(8, 8)], midb)
        accv = wscan(accv)
        w_s = jnp.max(accv)

        cp1.wait()
        cp2.wait()

        cbp = jnp.where(w_s == 0.0, eos_cbp, cbp_spec)

        @pl.when((w_s == 0.0) & (eos_cbp != cbp_spec))
        def _():
            build_idx(eos_cbp + 16)
            cp1, cp2 = fire_gather()
            cp1.wait()
            cp2.wait()

        @plsc.parallel_loop(0, seq, 1, unroll=4)
        def padd(s):
            for c in range(nchunk):
                sl = pl.ds(c * _L, _L)
                buf[s, sl] = buf[s, sl] + pos_t[s, sl]

        @pl.when(w_s > 0.0)
        def _():
            def mfix(base):
                @plsc.parallel_loop(0, 8, 1, unroll=2)
                def mloop(j):
                    s = cbp + base + j
                    for c in range(nchunk):
                        sl = pl.ds(c * _L, _L)
                        buf[s, sl] = midb[j, sl] + pos_t[s, sl]
            mfix(8)
            pltpu.sync_copy(ctx_hbm.at[b, pl.ds(0, 8)], midb)
            mfix(0)

        pltpu.async_copy(buf, out_hbm.at[b], sem_out)
        return carry

    lax.fori_loop(0, rows_per_w, row_body, 0)
    b_last = wid * rows_per_w + rows_per_w - 1
    pltpu.make_async_copy(buf, out_hbm.at[b_last], sem_out).wait()


@jax.jit
def _run(ctx_embeddings, ids_pack, token_table, position_table):
    B = ids_pack.shape[0]
    mesh = plsc.VectorSubcoreMesh(core_axis_name="c", subcore_axis_name="s")
    f = functools.partial(
        pl.kernel,
        out_type=jax.ShapeDtypeStruct((B, 77, 768), jnp.float32),
        mesh=mesh,
        scratch_types=[
            pltpu.VMEM((77, 768), jnp.float32),
            pltpu.VMEM((77, 768), jnp.float32),
            pltpu.VMEM((8, 768), jnp.float32),
            pltpu.VMEM((128,), jnp.int32),
            pltpu.VMEM((64,), jnp.int32),
            pltpu.VMEM((16,), jnp.int32),
            pltpu.SemaphoreType.DMA,
            pltpu.SemaphoreType.DMA,
        ],
        compiler_params=pltpu.CompilerParams(
            needs_layout_passes=False, use_tc_tiling_on_sc=False),
    )(_sc_body)
    return f(ctx_embeddings, ids_pack, token_table, position_table)


def kernel(ctx_embeddings, ctx_begin_pos, input_ids, token_table, position_table):
    B, S = input_ids.shape
    ids_pack = jnp.zeros((B, 128), jnp.int32)
    ids_pack = ids_pack.at[:, :S].set(input_ids.astype(jnp.int32))
    ids_pack = ids_pack.at[:, 96:112].set(
        ctx_begin_pos.astype(jnp.int32)[:, None])
    return _run(ctx_embeddings, ids_pack, token_table, position_table)


# speedup vs baseline: 1.0823x; 1.0823x over previous
import functools

import jax
import jax.numpy as jnp
from jax import lax
from jax.experimental import pallas as pl
from jax.experimental.pallas import tpu as pltpu
from jax.experimental.pallas import tpu_sc as plsc

_EOS = 49407
_L = 16
_NW = 32


def _worker_id():
    return lax.axis_index("s") * 2 + lax.axis_index("c")


def _sc_body(ctx_hbm, ids_hbm, tok_hbm, pos_hbm, out_hbm,
             pos_t, buf, midb, idsrow, idxl, idxl2, sem, sem_out):
    seq, emb = 77, 768
    nchunk = emb // _L
    rows_per_w = ids_hbm.shape[0] // _NW

    wid = _worker_id()

    pltpu.sync_copy(pos_hbm, pos_t)

    def build_idx(cbp16):
        idxl[pl.ds(0, _L)] = idsrow[pl.ds(0, _L)]
        for off in (16, 32, 48):
            a = idsrow[pl.ds(off, _L)]
            sh = idsrow[pl.ds(off - 16, _L)]
            posv = lax.iota(jnp.int32, _L) + off
            idxl[pl.ds(off, _L)] = jnp.where(posv < cbp16, a, sh)
        a = idsrow[pl.ds(61, _L)]
        sh = idsrow[pl.ds(45, _L)]
        posv = lax.iota(jnp.int32, _L) + 61
        idxl2[...] = jnp.where(posv < cbp16, a, sh)

    def fire_gather():
        cp1 = pltpu.async_copy(tok_hbm.at[idxl], buf.at[pl.ds(0, 64)], sem)
        cp2 = pltpu.async_copy(tok_hbm.at[idxl2], buf.at[pl.ds(61, _L)], sem)
        return cp1, cp2

    def row_body(i, carry):
        b = wid * rows_per_w + i
        pltpu.sync_copy(ids_hbm.at[b], idsrow)

        m = jnp.full((_L,), 127.0, jnp.float32)
        for k in range(5):
            v = idsrow[pl.ds(_L * k, _L)]
            posv = (lax.iota(jnp.int32, _L) + _L * k).astype(jnp.float32)
            m = jnp.minimum(m, jnp.where(v == _EOS, posv, 127.0))
        m_s = jnp.min(m)
        eos = jnp.where(m_s >= 127.0, 0, m_s.astype(jnp.int32))

        cbp_raw = jnp.max(idsrow[pl.ds(96, _L)].astype(jnp.float32)).astype(jnp.int32)
        cbp_spec = jnp.minimum(jnp.where(cbp_raw == -1, eos, cbp_raw), seq - 16)
        eos_cbp = jnp.minimum(eos, seq - 16)

        @pl.when(i > 0)
        def _():
            pltpu.make_async_copy(buf, out_hbm.at[b], sem_out).wait()

        build_idx(cbp_spec + 16)
        cp1, cp2 = fire_gather()

        def wscan(acc):
            def wloop(j, a):
                r = a
                for c in range(nchunk):
                    v = midb[j, pl.ds(c * _L, _L)]
                    r = jnp.maximum(r, jnp.where(v != 0.0, 1.0, 0.0))
                return r
            return lax.fori_loop(0, 8, wloop, acc)
        pltpu.sync_copy(ctx_hbm.at[b, pl.ds(0, 8)], midb)
        accv = wscan(jnp.zeros((_L,), jnp.float32))
        pltpu.sync_copy(ctx_hbm.at[b, pl.ds(8, 8)], midb)
        accv = wscan(accv)
        w_s = jnp.max(accv)

        cp1.wait()
        cp2.wait()

        cbp = jnp.where(w_s == 0.0, eos_cbp, cbp_spec)

        @pl.when((w_s == 0.0) & (eos_cbp != cbp_spec))
        def _():
            build_idx(eos_cbp + 16)
            cp1, cp2 = fire_gather()
            cp1.wait()
            cp2.wait()

        @plsc.parallel_loop(0, seq, 1, unroll=2)
        def padd(s):
            for c in range(nchunk):
                sl = pl.ds(c * _L, _L)
                buf[s, sl] = buf[s, sl] + pos_t[s, sl]

        @pl.when(w_s > 0.0)
        def _():
            def mfix(base):
                def mloop(j, _):
                    s = cbp + base + j
                    for c in range(nchunk):
                        sl = pl.ds(c * _L, _L)
                        buf[s, sl] = midb[j, sl] + pos_t[s, sl]
                    return 0
                lax.fori_loop(0, 8, mloop, 0)
            mfix(8)
            pltpu.sync_copy(ctx_hbm.at[b, pl.ds(0, 8)], midb)
            mfix(0)

        pltpu.async_copy(buf, out_hbm.at[b], sem_out)
        return carry

    lax.fori_loop(0, rows_per_w, row_body, 0)
    b_last = wid * rows_per_w + rows_per_w - 1
    pltpu.make_async_copy(buf, out_hbm.at[b_last], sem_out).wait()


@jax.jit
def _run(ctx_embeddings, ids_pack, token_table, position_table):
    B = ids_pack.shape[0]
    mesh = plsc.VectorSubcoreMesh(core_axis_name="c", subcore_axis_name="s")
    f = functools.partial(
        pl.kernel,
        out_type=jax.ShapeDtypeStruct((B, 77, 768), jnp.float32),
        mesh=mesh,
        scratch_types=[
            pltpu.VMEM((77, 768), jnp.float32),
            pltpu.VMEM((77, 768), jnp.float32),
            pltpu.VMEM((8, 768), jnp.float32),
            pltpu.VMEM((128,), jnp.int32),
            pltpu.VMEM((64,), jnp.int32),
            pltpu.VMEM((16,), jnp.int32),
            pltpu.SemaphoreType.DMA,
            pltpu.SemaphoreType.DMA,
        ],
        compiler_params=pltpu.CompilerParams(
            needs_layout_passes=False, use_tc_tiling_on_sc=False),
    )(_sc_body)
    return f(ctx_embeddings, ids_pack, token_table, position_table)


def kernel(ctx_embeddings, ctx_begin_pos, input_ids, token_table, position_table):
    B, S = input_ids.shape
    ids_pack = jnp.zeros((B, 128), jnp.int32)
    ids_pack = ids_pack.at[:, :S].set(input_ids.astype(jnp.int32))
    ids_pack = ids_pack.at[:, 96:112].set(
        ctx_begin_pos.astype(jnp.int32)[:, None])
    return _run(ctx_embeddings, ids_pack, token_table, position_table)
